# baseline (speedup 1.0000x reference)
import functools

import jax
import jax.numpy as jnp
from jax import lax
from jax.experimental import pallas as pl
from jax.experimental.pallas import tpu as pltpu
from jax.experimental.pallas import tpu_sc as plsc

B = 4096
F = 26
V = 1000
D = 128
DENSE = 13
EPS = 1e-5

NW = 32
RPW = B * F // NW
G = RPW // 128
GPF = B // 128

BT1 = 512
BT2 = 512


def _sc_gather(flat_idx, table, first_table):
    mesh = plsc.VectorSubcoreMesh(core_axis_name="c", subcore_axis_name="s")
    info = plsc.get_sparse_core_info()
    nc = info.num_cores

    @functools.partial(
        pl.kernel,
        mesh=mesh,
        out_type=(
            jax.ShapeDtypeStruct((F * B, D), jnp.float32),
            jax.ShapeDtypeStruct((NW, G, 128), jnp.float32),
        ),
        scratch_types=[
            pltpu.VMEM((G, 128), jnp.int32),
            pltpu.VMEM((128, D), jnp.float32),
            pltpu.VMEM((128, D), jnp.float32),
            pltpu.VMEM((128, D), jnp.float32),
            pltpu.VMEM((128, D), jnp.float32),
            pltpu.VMEM((128, D), jnp.float32),
            pltpu.VMEM((128, D), jnp.float32),
            pltpu.VMEM((G, 128), jnp.float32),
            pltpu.SemaphoreType.DMA,
            pltpu.SemaphoreType.DMA,
            pltpu.SemaphoreType.DMA,
            pltpu.SemaphoreType.DMA,
            pltpu.SemaphoreType.DMA,
            pltpu.SemaphoreType.DMA,
            pltpu.SemaphoreType.DMA,
            pltpu.SemaphoreType.DMA,
            pltpu.SemaphoreType.DMA,
            pltpu.SemaphoreType.DMA,
            pltpu.SemaphoreType.DMA,
            pltpu.SemaphoreType.DMA,
            pltpu.SemaphoreType.DMA,
        ],
    )
    def k(idx_hbm, tab_hbm, ft_hbm, emb_out, first_out,
          idx_v, r0, r1, r2, r3, r4, r5, f_v,
          gs0, gs1, gs2, gs3, gs4, gs5, os0, os1, os2, os3, os4, os5, fsem):
        wid = lax.axis_index("s") * nc + lax.axis_index("c")
        pltpu.sync_copy(idx_hbm.at[wid], idx_v)
        rs = (r0, r1, r2, r3, r4, r5)
        gss = (gs0, gs1, gs2, gs3, gs4, gs5)
        oss = (os0, os1, os2, os3, os4, os5)

        def dst(g):
            return emb_out.at[pl.ds(wid * RPW + g * 128, 128)]

        def gstart(j, g):
            pltpu.async_copy(tab_hbm.at[idx_v.at[g]], rs[j], gss[j])

        def gwait(j, g):
            pltpu.make_async_copy(tab_hbm.at[idx_v.at[g]], rs[j], gss[j]).wait()

        def ostart(j, g):
            pltpu.async_copy(rs[j], dst(g), oss[j])

        def owait(j, g):
            pltpu.make_async_copy(rs[j], dst(g), oss[j]).wait()

        def fstart(g):
            pltpu.async_copy(ft_hbm.at[idx_v.at[g]], f_v.at[g], fsem)

        NB = 6
        for j in range(NB):
            gstart(j, j)

        def hexa(q, carry):
            gq = NB * q
            for j in range(NB):
                gwait(j, gq + j)
                ostart(j, gq + j)
                fstart(gq + j)
            for j in range(NB):
                owait(j, gq + j)
                ng = gq + j + NB

                @pl.when(ng < G)
                def _():
                    gstart(j, ng)
            return carry

        lax.fori_loop(0, (G - 2) // NB, hexa, 0)
        for j, g in ((0, G - 2), (1, G - 1)):
            gwait(j, g)
            ostart(j, g)
            fstart(g)
            owait(j, g)

        def drain_f(g, carry):
            pltpu.make_async_copy(ft_hbm.at[idx_v.at[g]], f_v.at[g], fsem).wait()
            return carry

        lax.fori_loop(0, G, drain_f, 0)
        pltpu.sync_copy(f_v, first_out.at[wid])

    return k(flat_idx, table, first_table)


NA = B // BT2


def _bn_relu(zs, j, s, q, g, be):
    m = s[...] / B
    var = q[...] / B - m * m
    scale = g[...] * lax.rsqrt(var + EPS)
    shift = be[...] - m * scale
    zin = zs[pl.ds(j * BT2, BT2), :].astype(jnp.float32)
    return jnp.maximum(zin * scale + shift, 0.0)


def _acc_stats(i0, i, z, s, q):
    @pl.when(i == i0)
    def _():
        s[...] = jnp.zeros_like(s)
        q[...] = jnp.zeros_like(q)

    s[...] += jnp.sum(z, axis=0, keepdims=True)
    q[...] += jnp.sum(z * z, axis=0, keepdims=True)


def _all_body(emb_ref, dense_ref, wa_ref, wb_ref, b1_ref, g1_ref, be1_ref,
              w2_ref, b2_ref, g2_ref, be2_ref, w3_ref, b3_ref, g3_ref, be3_ref,
              w4_ref, b4_ref, f1_ref, wfd_ref, bfd_ref, bias_ref,
              out_ref,
              z1s, z2s, z3s, secs, s1s, q1s, s2s, q2s, s3s, q3s):
    i = pl.program_id(0)

    @pl.when(i < NA)
    def _():
        embs = [emb_ref[f] for f in range(F)]
        acc = embs[0]
        sqs = jnp.sum(embs[0] * embs[0], axis=1)
        for f in range(1, F):
            ef = embs[f]
            acc = acc + ef
            sqs = sqs + jnp.sum(ef * ef, axis=1)
        secs[pl.ds(i * BT2, BT2), :] = (
            0.5 * (jnp.sum(acc * acc, axis=1) - sqs))[:, None]

        x = jnp.concatenate([e.astype(jnp.bfloat16) for e in embs], axis=1)
        nt = (((1,), (1,)), ((), ()))
        z = lax.dot_general(x, wa_ref[...], nt,
                            preferred_element_type=jnp.float32)
        z = z + lax.dot_general(dense_ref[...].astype(jnp.bfloat16),
                                wb_ref[...], nt,
                                preferred_element_type=jnp.float32)
        z = z + b1_ref[...]
        z1s[pl.ds(i * BT2, BT2), :] = z.astype(jnp.bfloat16)
        _acc_stats(0, i, z, s1s, q1s)

    @pl.when(jnp.logical_and(i >= NA, i < 2 * NA))
    def _():
        j = i - NA
        x = _bn_relu(z1s, j, s1s, q1s, g1_ref, be1_ref)
        z = lax.dot_general(x.astype(jnp.bfloat16), w2_ref[...],
                            (((1,), (1,)), ((), ())),
                            preferred_element_type=jnp.float32) + b2_ref[...]
        z2s[pl.ds(j * BT2, BT2), :] = z.astype(jnp.bfloat16)
        _acc_stats(NA, i, z, s2s, q2s)

    @pl.when(jnp.logical_and(i >= 2 * NA, i < 3 * NA))
    def _():
        j = i - 2 * NA
        x = _bn_relu(z2s, j, s2s, q2s, g2_ref, be2_ref)
        z = lax.dot_general(x.astype(jnp.bfloat16), w3_ref[...],
                            (((1,), (1,)), ((), ())),
                            preferred_element_type=jnp.float32) + b3_ref[...]
        z3s[pl.ds(j * BT2, BT2), :] = z.astype(jnp.bfloat16)
        _acc_stats(2 * NA, i, z, s3s, q3s)

    @pl.when(i >= 3 * NA)
    def _():
        j = i - 3 * NA
        x = _bn_relu(z3s, j, s3s, q3s, g3_ref, be3_ref)
        dnn = jnp.sum(x * w4_ref[...], axis=1, keepdims=True) + b4_ref[...]
        first = (jnp.sum(f1_ref[...], axis=1, keepdims=True)
                 + jnp.sum(dense_ref[...] * wfd_ref[...], axis=1, keepdims=True)
                 + bfd_ref[...])
        out_ref[...] = jax.nn.sigmoid(
            dnn + first + secs[pl.ds(j * BT2, BT2), :] + bias_ref[...])


def _row(x):
    return x.reshape(1, -1)


def kernel(sparse_features, dense_features, emb_first, w_fd, b_fd, emb_tables,
           W1, b1, g1, be1, W2, b2, g2, be2, W3, b3, g3, be3, W4, b4, bias):
    f32 = jnp.float32
    bf16 = jnp.bfloat16

    flat_idx = (sparse_features.T
                + (jnp.arange(F, dtype=jnp.int32) * V)[:, None]).reshape(NW, G, 128)
    table = emb_tables.reshape(F * V, D)
    ftab = emb_first.reshape(F * V)

    emb_flat, firsts = _sc_gather(flat_idx, table, ftab)
    emb3d = emb_flat.reshape(F, B, D)
    firsts2d = firsts.reshape(F, B).T

    w1a = W1[:, :F * D].astype(bf16)
    w1b = W1[:, F * D:].astype(bf16)
    w2b = W2.astype(bf16)
    w3b = W3.astype(bf16)

    h1, h2, h3 = W1.shape[0], W2.shape[0], W3.shape[0]

    def c00(i):
        return (0, 0)

    def phase_a(i):
        return (jnp.where(i < NA, i, jnp.maximum(i - 3 * NA, 0)), 0)

    def phase_d(i):
        return (jnp.maximum(i - 3 * NA, 0), 0)

    out = pl.pallas_call(
        _all_body,
        grid=(4 * NA,),
        in_specs=[
            pl.BlockSpec((F, BT2, D), lambda i: (0, jnp.minimum(i, NA - 1), 0)),
            pl.BlockSpec((BT2, DENSE), phase_a),
            pl.BlockSpec((h1, F * D), c00),
            pl.BlockSpec((h1, DENSE), c00),
            pl.BlockSpec((1, h1), c00),
            pl.BlockSpec((1, h1), c00),
            pl.BlockSpec((1, h1), c00),
            pl.BlockSpec((h2, h1), c00),
            pl.BlockSpec((1, h2), c00),
            pl.BlockSpec((1, h2), c00),
            pl.BlockSpec((1, h2), c00),
            pl.BlockSpec((h3, h2), c00),
            pl.BlockSpec((1, h3), c00),
            pl.BlockSpec((1, h3), c00),
            pl.BlockSpec((1, h3), c00),
            pl.BlockSpec((1, h3), c00),
            pl.BlockSpec((1, 1), c00),
            pl.BlockSpec((BT2, F), phase_d),
            pl.BlockSpec((1, DENSE), c00),
            pl.BlockSpec((1, 1), c00),
            pl.BlockSpec((1, 1), c00),
        ],
        out_specs=pl.BlockSpec((BT2, 1), phase_d),
        out_shape=jax.ShapeDtypeStruct((B, 1), f32),
        scratch_shapes=[
            pltpu.VMEM((B, h1), bf16),
            pltpu.VMEM((B, h2), bf16),
            pltpu.VMEM((B, h3), bf16),
            pltpu.VMEM((B, 1), f32),
            pltpu.VMEM((1, h1), f32),
            pltpu.VMEM((1, h1), f32),
            pltpu.VMEM((1, h2), f32),
            pltpu.VMEM((1, h2), f32),
            pltpu.VMEM((1, h3), f32),
            pltpu.VMEM((1, h3), f32),
        ],
    )(emb3d, dense_features, w1a, w1b, _row(b1), _row(g1), _row(be1),
      w2b, _row(b2), _row(g2), _row(be2), w3b, _row(b3), _row(g3), _row(be3),
      _row(W4), b4.reshape(1, 1), firsts2d, w_fd, b_fd.reshape(1, 1),
      bias.reshape(1, 1))

    return out.reshape(B)


# speedup vs baseline: 1.0119x; 1.0119x over previous
import functools

import jax
import jax.numpy as jnp
from jax import lax
from jax.experimental import pallas as pl
from jax.experimental.pallas import tpu as pltpu
from jax.experimental.pallas import tpu_sc as plsc

B = 4096
F = 26
V = 1000
D = 128
DENSE = 13
EPS = 1e-5

NW = 32
RPW = B * F // NW
G = RPW // 128
GPF = B // 128

BT1 = 512
BT2 = 512


def _sc_gather(flat_idx, table, first_table):
    mesh = plsc.VectorSubcoreMesh(core_axis_name="c", subcore_axis_name="s")
    info = plsc.get_sparse_core_info()
    nc = info.num_cores

    @functools.partial(
        pl.kernel,
        mesh=mesh,
        out_type=(
            jax.ShapeDtypeStruct((F * B, D), jnp.float32),
            jax.ShapeDtypeStruct((NW, G, 128), jnp.float32),
        ),
        scratch_types=[
            pltpu.VMEM((G, 128), jnp.int32),
            pltpu.VMEM((128, D), jnp.float32),
            pltpu.VMEM((128, D), jnp.float32),
            pltpu.VMEM((128, D), jnp.float32),
            pltpu.VMEM((128, D), jnp.float32),
            pltpu.VMEM((128, D), jnp.float32),
            pltpu.VMEM((128, D), jnp.float32),
            pltpu.VMEM((G, 128), jnp.float32),
            pltpu.SemaphoreType.DMA,
            pltpu.SemaphoreType.DMA,
            pltpu.SemaphoreType.DMA,
            pltpu.SemaphoreType.DMA,
            pltpu.SemaphoreType.DMA,
            pltpu.SemaphoreType.DMA,
            pltpu.SemaphoreType.DMA,
            pltpu.SemaphoreType.DMA,
            pltpu.SemaphoreType.DMA,
            pltpu.SemaphoreType.DMA,
            pltpu.SemaphoreType.DMA,
            pltpu.SemaphoreType.DMA,
            pltpu.SemaphoreType.DMA,
        ],
    )
    def k(idx_hbm, tab_hbm, ft_hbm, emb_out, first_out,
          idx_v, r0, r1, r2, r3, r4, r5, f_v,
          gs0, gs1, gs2, gs3, gs4, gs5, os0, os1, os2, os3, os4, os5, fsem):
        wid = lax.axis_index("s") * nc + lax.axis_index("c")
        pltpu.sync_copy(idx_hbm.at[wid], idx_v)
        rs = (r0, r1, r2, r3, r4, r5)
        gss = (gs0, gs1, gs2, gs3, gs4, gs5)
        oss = (os0, os1, os2, os3, os4, os5)

        def dst(g):
            return emb_out.at[pl.ds(wid * RPW + g * 128, 128)]

        def gstart(j, g):
            pltpu.async_copy(tab_hbm.at[idx_v.at[g]], rs[j], gss[j])

        def gwait(j, g):
            pltpu.make_async_copy(tab_hbm.at[idx_v.at[g]], rs[j], gss[j]).wait()

        def ostart(j, g):
            pltpu.async_copy(rs[j], dst(g), oss[j])

        def owait(j, g):
            pltpu.make_async_copy(rs[j], dst(g), oss[j]).wait()

        def fstart(g):
            pltpu.async_copy(ft_hbm.at[idx_v.at[g]], f_v.at[g], fsem)

        NB = 6
        for j in range(NB):
            gstart(j, j)

        def hexa(q, carry):
            gq = NB * q
            for j in range(NB):
                gwait(j, gq + j)
                ostart(j, gq + j)
                fstart(gq + j)
            for j in range(NB):
                owait(j, gq + j)
                ng = gq + j + NB

                @pl.when(ng < G)
                def _():
                    gstart(j, ng)
            return carry

        lax.fori_loop(0, (G - 2) // NB, hexa, 0)
        for j, g in ((0, G - 2), (1, G - 1)):
            gwait(j, g)
            ostart(j, g)
            fstart(g)
            owait(j, g)

        def drain_f(g, carry):
            pltpu.make_async_copy(ft_hbm.at[idx_v.at[g]], f_v.at[g], fsem).wait()
            return carry

        lax.fori_loop(0, G, drain_f, 0)
        pltpu.sync_copy(f_v, first_out.at[wid])

    return k(flat_idx, table, first_table)


NA = B // BT2


def _bn_relu(zs, j, s, q, g, be):
    m = s[...] / B
    var = q[...] / B - m * m
    scale = g[...] * lax.rsqrt(var + EPS)
    shift = be[...] - m * scale
    zin = zs[pl.ds(j * BT2, BT2), :].astype(jnp.float32)
    return jnp.maximum(zin * scale + shift, 0.0)


def _acc_stats(i0, i, z, s, q):
    @pl.when(i == i0)
    def _():
        s[...] = jnp.zeros_like(s)
        q[...] = jnp.zeros_like(q)

    s[...] += jnp.sum(z, axis=0, keepdims=True)
    q[...] += jnp.sum(z * z, axis=0, keepdims=True)


def _all_body(emb_ref, dense_ref, wa_ref, wb_ref, b1_ref, g1_ref, be1_ref,
              w2_ref, b2_ref, g2_ref, be2_ref, w3_ref, b3_ref, g3_ref, be3_ref,
              w4_ref, b4_ref, f1_ref, wfd_ref, bfd_ref, bias_ref,
              out_ref,
              z1s, z2s, z3s, secs, s1s, q1s, s2s, q2s, s3s, q3s):
    i = pl.program_id(0)

    @pl.when(i < NA)
    def _():
        embs = [emb_ref[f] for f in range(F)]
        acc = embs[0]
        sqs = jnp.sum(embs[0] * embs[0], axis=1)
        for f in range(1, F):
            ef = embs[f]
            acc = acc + ef
            sqs = sqs + jnp.sum(ef * ef, axis=1)
        secs[pl.ds(i * BT2, BT2), :] = (
            0.5 * (jnp.sum(acc * acc, axis=1) - sqs))[:, None]

        x = jnp.concatenate([e.astype(jnp.bfloat16) for e in embs], axis=1)
        z = jnp.dot(x, wa_ref[...], preferred_element_type=jnp.float32)
        z = z + jnp.dot(dense_ref[...].astype(jnp.bfloat16), wb_ref[...],
                        preferred_element_type=jnp.float32)
        z = z + b1_ref[...]
        z1s[pl.ds(i * BT2, BT2), :] = z.astype(jnp.bfloat16)
        _acc_stats(0, i, z, s1s, q1s)

    @pl.when(jnp.logical_and(i >= NA, i < 2 * NA))
    def _():
        j = i - NA
        x = _bn_relu(z1s, j, s1s, q1s, g1_ref, be1_ref)
        z = jnp.dot(x.astype(jnp.bfloat16), w2_ref[...],
                    preferred_element_type=jnp.float32) + b2_ref[...]
        z2s[pl.ds(j * BT2, BT2), :] = z.astype(jnp.bfloat16)
        _acc_stats(NA, i, z, s2s, q2s)

    @pl.when(jnp.logical_and(i >= 2 * NA, i < 3 * NA))
    def _():
        j = i - 2 * NA
        x = _bn_relu(z2s, j, s2s, q2s, g2_ref, be2_ref)
        z = jnp.dot(x.astype(jnp.bfloat16), w3_ref[...],
                    preferred_element_type=jnp.float32) + b3_ref[...]
        z3s[pl.ds(j * BT2, BT2), :] = z.astype(jnp.bfloat16)
        _acc_stats(2 * NA, i, z, s3s, q3s)

    @pl.when(i >= 3 * NA)
    def _():
        j = i - 3 * NA
        x = _bn_relu(z3s, j, s3s, q3s, g3_ref, be3_ref)
        dnn = jnp.sum(x * w4_ref[...], axis=1, keepdims=True) + b4_ref[...]
        first = (jnp.sum(f1_ref[...], axis=1, keepdims=True)
                 + jnp.sum(dense_ref[...] * wfd_ref[...], axis=1, keepdims=True)
                 + bfd_ref[...])
        out_ref[...] = jax.nn.sigmoid(
            dnn + first + secs[pl.ds(j * BT2, BT2), :] + bias_ref[...])


def _row(x):
    return x.reshape(1, -1)


def kernel(sparse_features, dense_features, emb_first, w_fd, b_fd, emb_tables,
           W1, b1, g1, be1, W2, b2, g2, be2, W3, b3, g3, be3, W4, b4, bias):
    f32 = jnp.float32
    bf16 = jnp.bfloat16

    flat_idx = (sparse_features.T
                + (jnp.arange(F, dtype=jnp.int32) * V)[:, None]).reshape(NW, G, 128)
    table = emb_tables.reshape(F * V, D)
    ftab = emb_first.reshape(F * V)

    emb_flat, firsts = _sc_gather(flat_idx, table, ftab)
    emb3d = emb_flat.reshape(F, B, D)
    firsts2d = firsts.reshape(F, B).T

    w1aT = W1[:, :F * D].T.astype(bf16)
    w1bT = W1[:, F * D:].T.astype(bf16)
    w2T = W2.T.astype(bf16)
    w3T = W3.T.astype(bf16)

    h1, h2, h3 = W1.shape[0], W2.shape[0], W3.shape[0]

    def c00(i):
        return (0, 0)

    def phase_a(i):
        return (jnp.where(i < NA, i, jnp.maximum(i - 3 * NA, 0)), 0)

    def phase_d(i):
        return (jnp.maximum(i - 3 * NA, 0), 0)

    out = pl.pallas_call(
        _all_body,
        grid=(4 * NA,),
        in_specs=[
            pl.BlockSpec((F, BT2, D), lambda i: (0, jnp.minimum(i, NA - 1), 0)),
            pl.BlockSpec((BT2, DENSE), phase_a),
            pl.BlockSpec((F * D, h1), c00),
            pl.BlockSpec((DENSE, h1), c00),
            pl.BlockSpec((1, h1), c00),
            pl.BlockSpec((1, h1), c00),
            pl.BlockSpec((1, h1), c00),
            pl.BlockSpec((h1, h2), c00),
            pl.BlockSpec((1, h2), c00),
            pl.BlockSpec((1, h2), c00),
            pl.BlockSpec((1, h2), c00),
            pl.BlockSpec((h2, h3), c00),
            pl.BlockSpec((1, h3), c00),
            pl.BlockSpec((1, h3), c00),
            pl.BlockSpec((1, h3), c00),
            pl.BlockSpec((1, h3), c00),
            pl.BlockSpec((1, 1), c00),
            pl.BlockSpec((BT2, F), phase_d),
            pl.BlockSpec((1, DENSE), c00),
            pl.BlockSpec((1, 1), c00),
            pl.BlockSpec((1, 1), c00),
        ],
        out_specs=pl.BlockSpec((BT2, 1), phase_d),
        out_shape=jax.ShapeDtypeStruct((B, 1), f32),
        scratch_shapes=[
            pltpu.VMEM((B, h1), bf16),
            pltpu.VMEM((B, h2), bf16),
            pltpu.VMEM((B, h3), bf16),
            pltpu.VMEM((B, 1), f32),
            pltpu.VMEM((1, h1), f32),
            pltpu.VMEM((1, h1), f32),
            pltpu.VMEM((1, h2), f32),
            pltpu.VMEM((1, h2), f32),
            pltpu.VMEM((1, h3), f32),
            pltpu.VMEM((1, h3), f32),
        ],
    )(emb3d, dense_features, w1aT, w1bT, _row(b1), _row(g1), _row(be1),
      w2T, _row(b2), _row(g2), _row(be2), w3T, _row(b3), _row(g3), _row(be3),
      _row(W4), b4.reshape(1, 1), firsts2d, w_fd, b_fd.reshape(1, 1),
      bias.reshape(1, 1))

    return out.reshape(B)
